# Initial kernel scaffold; baseline (speedup 1.0000x reference)
import jax
import jax.numpy as jnp
from jax.experimental import pallas as pl


def kernel(hidden_states, gate_w, w1, w2):
    raise NotImplementedError("write your pallas kernel here")


# speedup vs baseline: 9.6302x; 9.6302x over previous
import functools

import jax
import jax.numpy as jnp
from jax import lax
from jax.experimental import pallas as pl
from jax.experimental.pallas import tpu as pltpu
from jax.experimental.pallas import tpu_sc as plsc

S = 8192
H = 768
E = 64
F = 1536
TB = 256
T = 256
NB = S // T + E
NPAD = NB * T

NW = 32
ROWS_W = S // NW
CHUNK = 128


def _router_body(x_ref, gw_ref, eid_ref, rank_ref, cnt_ref):
    i = pl.program_id(0)

    @pl.when(i == 0)
    def _():
        cnt_ref[...] = jnp.zeros((8, 128), jnp.int32)

    logits = jnp.dot(x_ref[...], gw_ref[...], preferred_element_type=jnp.float32)
    mx = jnp.max(logits, axis=1, keepdims=True)
    lane = lax.broadcasted_iota(jnp.int32, (TB, E), 1)
    eid = jnp.min(jnp.where(logits == mx, lane, E), axis=1, keepdims=True)
    oh = jnp.where(eid == lane, 1.0, 0.0)
    r0 = lax.broadcasted_iota(jnp.int32, (TB, TB), 0)
    r1 = lax.broadcasted_iota(jnp.int32, (TB, TB), 1)
    tril = jnp.where(r1 < r0, 1.0, 0.0)
    excl = jnp.dot(tril, oh, preferred_element_type=jnp.float32)
    prior = cnt_ref[0:1, 0:E].astype(jnp.float32)
    rank = jnp.sum(oh * (excl + prior), axis=1, keepdims=True)
    cnt_ref[0:1, 0:E] = (prior + jnp.sum(oh, axis=0, keepdims=True)).astype(jnp.int32)
    eid_ref[0] = eid
    rank_ref[0] = rank.astype(jnp.int32)


def _bin_body(cnt_ref, eid_ref, rank_ref, pos_ref, meta_ref):
    i = pl.program_id(0)
    cnt = cnt_ref[0:1, :]
    padded = ((cnt + (T - 1)) // T) * T
    padf = padded.astype(jnp.float32)
    r0 = lax.broadcasted_iota(jnp.int32, (128, 128), 0)
    r1 = lax.broadcasted_iota(jnp.int32, (128, 128), 1)
    incl = jnp.where(r0 <= r1, 1.0, 0.0)
    cum = jnp.dot(padf, incl, preferred_element_type=jnp.float32)
    poff = cum - padf

    lane = lax.broadcasted_iota(jnp.int32, (TB, E), 1)
    oh = eid_ref[0] == lane
    posf = jnp.sum(jnp.where(oh, poff[0:1, 0:E], 0.0), axis=1, keepdims=True)
    pos_ref[0] = posf.astype(jnp.int32) + rank_ref[0]

    @pl.when(i == 0)
    def _():
        total = cum[0:1, 127:128]
        eye = jnp.where(r0 == r1, 1.0, 0.0)
        cum_col = lax.dot_general(eye, cum, (((1,), (1,)), ((), ())),
                                  preferred_element_type=jnp.float32)
        jblk = lax.broadcasted_iota(jnp.int32, (1, 128), 1).astype(jnp.float32)
        bstart = jblk * float(T)
        s = jnp.minimum(bstart, total - float(T))
        eob = jnp.sum(jnp.where(cum_col <= s, 1.0, 0.0), axis=0, keepdims=True)
        nblk = total * (1.0 / float(T))
        rowblk = jnp.minimum(jblk, nblk - 1.0)
        meta_ref[0:1, :] = eob.astype(jnp.int32)
        meta_ref[1:2, :] = rowblk.astype(jnp.int32)


def _ffn_body(rb_ref, eb_ref, x_ref, w1_ref, w2_ref, o_ref):
    del rb_ref, eb_ref
    h = jnp.maximum(
        jnp.dot(x_ref[...], w1_ref[0], preferred_element_type=jnp.float32), 0.0)
    o_ref[...] = jnp.dot(h, w2_ref[0], preferred_element_type=jnp.float32)


def _dispatch_body(flat_hbm, pos_hbm, xs_hbm, idx_v, rows_v, sem):
    wid = lax.axis_index("s") * 2 + lax.axis_index("c")
    base = wid * ROWS_W
    for ch in range(ROWS_W // CHUNK):
        off = base + ch * CHUNK
        pltpu.sync_copy(flat_hbm.at[pl.ds(off, CHUNK)], rows_v)
        pltpu.sync_copy(pos_hbm.at[pl.ds(off, CHUNK)], idx_v)
        pltpu.async_copy(rows_v, xs_hbm.at[idx_v], sem).wait()


def _combine_body(ys_hbm, pos_hbm, out_hbm, idx_v, rows_v, sem):
    wid = lax.axis_index("s") * 2 + lax.axis_index("c")
    base = wid * ROWS_W
    for ch in range(ROWS_W // CHUNK):
        off = base + ch * CHUNK
        pltpu.sync_copy(pos_hbm.at[pl.ds(off, CHUNK)], idx_v)
        pltpu.async_copy(ys_hbm.at[idx_v], rows_v, sem).wait()
        pltpu.sync_copy(rows_v, out_hbm.at[pl.ds(off, CHUNK)])


def _sc_scratch():
    return [
        pltpu.VMEM((CHUNK,), jnp.int32),
        pltpu.VMEM((CHUNK, H), jnp.float32),
        pltpu.SemaphoreType.DMA,
    ]


@functools.cache
def _get_dispatch():
    mesh = plsc.VectorSubcoreMesh(core_axis_name="c", subcore_axis_name="s")
    return pl.kernel(
        _dispatch_body,
        out_type=jax.ShapeDtypeStruct((NPAD, H), jnp.float32),
        mesh=mesh,
        scratch_types=_sc_scratch(),
    )


@functools.cache
def _get_combine():
    mesh = plsc.VectorSubcoreMesh(core_axis_name="c", subcore_axis_name="s")
    return pl.kernel(
        _combine_body,
        out_type=jax.ShapeDtypeStruct((S, H), jnp.float32),
        mesh=mesh,
        scratch_types=_sc_scratch(),
    )


def kernel(hidden_states, gate_w, w1, w2):
    b, s, h = hidden_states.shape
    flat = hidden_states.reshape(s, h)

    eid, rank, counts = pl.pallas_call(
        _router_body,
        grid=(S // TB,),
        in_specs=[
            pl.BlockSpec((TB, H), lambda i: (i, 0)),
            pl.BlockSpec((H, E), lambda i: (0, 0)),
        ],
        out_specs=[
            pl.BlockSpec((1, TB, 1), lambda i: (i, 0, 0)),
            pl.BlockSpec((1, TB, 1), lambda i: (i, 0, 0)),
            pl.BlockSpec((8, 128), lambda i: (0, 0)),
        ],
        out_shape=[
            jax.ShapeDtypeStruct((S // TB, TB, 1), jnp.int32),
            jax.ShapeDtypeStruct((S // TB, TB, 1), jnp.int32),
            jax.ShapeDtypeStruct((8, 128), jnp.int32),
        ],
    )(flat, gate_w)

    pos, meta = pl.pallas_call(
        _bin_body,
        grid=(S // TB,),
        in_specs=[
            pl.BlockSpec((8, 128), lambda i: (0, 0)),
            pl.BlockSpec((1, TB, 1), lambda i: (i, 0, 0)),
            pl.BlockSpec((1, TB, 1), lambda i: (i, 0, 0)),
        ],
        out_specs=[
            pl.BlockSpec((1, TB, 1), lambda i: (i, 0, 0)),
            pl.BlockSpec((8, 128), lambda i: (0, 0)),
        ],
        out_shape=[
            jax.ShapeDtypeStruct((S // TB, TB, 1), jnp.int32),
            jax.ShapeDtypeStruct((8, 128), jnp.int32),
        ],
    )(counts, eid, rank)

    pos_flat = pos.reshape(s)
    eob = meta[0, :NB]
    rowblk = meta[1, :NB]

    xs = _get_dispatch()(flat, pos_flat)

    ys = pl.pallas_call(
        _ffn_body,
        grid_spec=pltpu.PrefetchScalarGridSpec(
            num_scalar_prefetch=2,
            grid=(NB,),
            in_specs=[
                pl.BlockSpec((T, H), lambda i, rb, eb: (rb[i], 0)),
                pl.BlockSpec((1, H, F), lambda i, rb, eb: (eb[i], 0, 0)),
                pl.BlockSpec((1, F, H), lambda i, rb, eb: (eb[i], 0, 0)),
            ],
            out_specs=pl.BlockSpec((T, H), lambda i, rb, eb: (rb[i], 0)),
        ),
        out_shape=jax.ShapeDtypeStruct((NPAD, H), jnp.float32),
    )(rowblk, eob, xs, w1, w2)

    out = _get_combine()(ys, pos_flat)
    return out.reshape(b, s, h)
